# baseline (speedup 1.0000x reference)
import functools

import jax
import jax.numpy as jnp
from jax import lax
from jax.experimental import pallas as pl
from jax.experimental.pallas import tpu as pltpu, tpu_sc as plsc

N = 10000
E = 320000
D = 128
DOUT = 64
EPS = 1e-5

NC = 2
NS = 16
NW = NC * NS
NPAD = 10240
RT = NPAD // NS
EW = E // NW
K = 80
NCH = EW // K
DW = 16
DH2 = D // 2


def _sc_agg(with_deg):
  mesh = plsc.VectorSubcoreMesh(core_axis_name="c", subcore_axis_name="s")
  out_type = [jax.ShapeDtypeStruct((NC, 2, NS, RT, DH2), jnp.float32)]
  if with_deg:
    out_type.append(jax.ShapeDtypeStruct((NC, NS, RT, DW), jnp.float32))
  scratch = [
      pltpu.VMEM((NCH, K), jnp.int32),
      pltpu.VMEM((NCH, K), jnp.int32),
      pltpu.VMEM((NCH, K), jnp.int32),
      pltpu.VMEM((K, DH2), jnp.float32),
      pltpu.VMEM((K, DH2), jnp.float32),
      pltpu.VMEM((K, DW), jnp.float32),
      pltpu.VMEM_SHARED((NPAD, DH2), jnp.float32),
      pltpu.VMEM_SHARED((NPAD, DW), jnp.float32),
      pltpu.SemaphoreType.DMA,
      pltpu.SemaphoreType.DMA,
      pltpu.SemaphoreType.DMA,
      pltpu.SemaphoreType.DMA,
      pltpu.SemaphoreType.DMA,
  ]
  if not with_deg:
    scratch = [sc for i, sc in enumerate(scratch) if i not in (5, 7, 12)]

  def body(h_hbm, src_hbm, dst_hbm, agg_out, *rest):
    if with_deg:
      (deg_out, srcb, srcb2, dstb, row0, row1, ones16, aggsh, degsh,
       gsem0, gsem1, ssem0, ssem1, dsem) = rest
    else:
      (srcb, srcb2, dstb, row0, row1, aggsh, gsem0, gsem1, ssem0,
       ssem1) = rest
    cid = lax.axis_index("c")
    sid = lax.axis_index("s")
    wid = sid * NC + cid
    tb = sid * RT

    pltpu.sync_copy(src_hbm.at[wid], srcb)
    pltpu.sync_copy(dst_hbm.at[wid], dstb)

    rows = (row0, row1)
    gsems = (gsem0, gsem1)
    ssems = (ssem0, ssem1)

    def gather(j, b):
      pltpu.async_copy(h_hbm.at[srcb2.at[j]], rows[b], gsems[b])

    def gwait(j, b):
      pltpu.make_async_copy(h_hbm.at[srcb2.at[j]], rows[b], gsems[b]).wait()

    def scat(j, b):
      pltpu.async_copy(rows[b], aggsh.at[dstb.at[j]], ssems[b], add=True)

    def swait(j, b):
      pltpu.make_async_copy(rows[b], aggsh.at[dstb.at[j]], ssems[b]).wait()

    for half in range(2):
      def trow(j, _):
        for l in range(K // 16):
          v = srcb[j, pl.ds(l * 16, 16)]
          srcb2[j, pl.ds(l * 16, 16)] = v * 2 + half
        return 0
      lax.fori_loop(0, NCH, trow, 0)

      def zrow(i, _):
        for j in range(DH2 // 16):
          row0[i, pl.ds(j * 16, 16)] = jnp.zeros((16,), jnp.float32)
        return 0
      lax.fori_loop(0, K, zrow, 0)
      for j in range(RT // K):
        pltpu.sync_copy(row0, aggsh.at[pl.ds(tb + j * K, K)])
      if with_deg and half == 0:
        def z16row(i, _):
          ones16[i, :] = jnp.zeros((DW,), jnp.float32)
          return 0
        lax.fori_loop(0, K, z16row, 0)
        for j in range(RT // K):
          pltpu.sync_copy(ones16, degsh.at[pl.ds(tb + j * K, K)])
        def frow(i, _):
          ones16[i, :] = jnp.ones((DW,), jnp.float32)
          return 0
        lax.fori_loop(0, K, frow, 0)

      plsc.subcore_barrier()

      gather(0, 0)
      gwait(0, 0)
      scat(0, 0)
      if with_deg and half == 0:
        pltpu.async_copy(ones16, degsh.at[dstb.at[0]], dsem, add=True)
      gather(1, 1)

      def step(t, _):
        for b2 in range(2):
          j = 2 * t + 1 + b2
          b = (1 + b2) % 2
          nb = (b + 1) % 2
          gwait(j, b)
          scat(j, b)
          if with_deg and half == 0:
            pltpu.async_copy(ones16, degsh.at[dstb.at[j]], dsem, add=True)
          swait(j - 1, nb)

          @pl.when(j < NCH - 1)
          def _():
            gather(j + 1, nb)

          if with_deg and half == 0:
            pltpu.make_async_copy(ones16, degsh.at[dstb.at[j - 1]], dsem).wait()
        return 0
      lax.fori_loop(0, (NCH - 1) // 2, step, 0)
      swait(NCH - 1, (NCH - 1) % 2)
      if with_deg and half == 0:
        pltpu.make_async_copy(ones16, degsh.at[dstb.at[NCH - 1]], dsem).wait()

      plsc.subcore_barrier()

      pltpu.sync_copy(aggsh.at[pl.ds(tb, RT)], agg_out.at[cid].at[half].at[sid])
      if with_deg and half == 0:
        pltpu.sync_copy(degsh.at[pl.ds(tb, RT)], deg_out.at[cid].at[sid])

  return pl.kernel(
      body, out_type=out_type, mesh=mesh, scratch_types=scratch,
      compiler_params=pltpu.CompilerParams(use_tc_tiling_on_sc=False))


def _tc_dense(do, relu, normalize):
  BN = 512
  grid = NPAD // BN

  def body(aggp_ref, degp_ref, h_ref, wl_ref, wr_ref, bl_ref, sg_ref,
           bb_ref, out_ref):
    deg = degp_ref[0, :, 0:1] + degp_ref[1, :, 0:1]
    r = 1.0 / jnp.maximum(deg, 1.0)
    agg = jnp.concatenate(
        [aggp_ref[0, 0] + aggp_ref[1, 0], aggp_ref[0, 1] + aggp_ref[1, 1]],
        axis=1) * r
    z = lax.dot_general(agg, wl_ref[...], (((1,), (1,)), ((), ())),
                        preferred_element_type=jnp.float32)
    z = z + lax.dot_general(h_ref[...], wr_ref[...], (((1,), (1,)), ((), ())),
                            preferred_element_type=jnp.float32)
    z = (z + bl_ref[...]) * sg_ref[...] + bb_ref[...]
    if relu:
      z = jnp.maximum(z, 0.0)
    if normalize:
      nrm = jnp.sqrt(jnp.sum(z * z, axis=1, keepdims=True))
      z = z / jnp.maximum(nrm, 1e-12)
    out_ref[...] = z

  return pl.pallas_call(
      body,
      grid=(grid,),
      in_specs=[
          pl.BlockSpec((NC, 2, BN, DH2), lambda i: (0, 0, i, 0)),
          pl.BlockSpec((NC, BN, DW), lambda i: (0, i, 0)),
          pl.BlockSpec((BN, D), lambda i: (i, 0)),
          pl.BlockSpec((do, D), lambda i: (0, 0)),
          pl.BlockSpec((do, D), lambda i: (0, 0)),
          pl.BlockSpec((1, do), lambda i: (0, 0)),
          pl.BlockSpec((1, do), lambda i: (0, 0)),
          pl.BlockSpec((1, do), lambda i: (0, 0)),
      ],
      out_specs=pl.BlockSpec((BN, do), lambda i: (i, 0)),
      out_shape=jax.ShapeDtypeStruct((NPAD, do), jnp.float32),
  )


_sc_agg_deg = _sc_agg(with_deg=True)
_sc_agg_only = _sc_agg(with_deg=False)
_tc1 = _tc_dense(D, relu=True, normalize=False)
_tc2 = _tc_dense(D, relu=True, normalize=False)
_tc3 = _tc_dense(DOUT, relu=False, normalize=True)


def kernel(x, edge_index, Wl1, bl1, Wr1, g1, b1, Wl2, bl2, Wr2, g2, b2,
           Wl3, bl3, Wr3, g3, b3):
  src = edge_index[0].reshape(NW, NCH, K)
  dst = edge_index[1].reshape(NW, NCH, K)
  xp = jnp.pad(x, ((0, NPAD - N), (0, 0)))

  t = 1.0 / jnp.sqrt(jnp.float32(1.0 + EPS))
  sg1, sg2, sg3 = g1 * t, g2 * t, g3 * t

  agg1, deg = _sc_agg_deg(xp.reshape(2 * NPAD, DH2), src, dst)
  agg1 = agg1.reshape(NC, 2, NPAD, DH2)
  degp = deg.reshape(NC, NPAD, DW)
  h1 = _tc1(agg1, degp, xp, Wl1, Wr1, bl1.reshape(1, D), sg1.reshape(1, D),
            b1.reshape(1, D))

  (agg2,) = _sc_agg_only(h1.reshape(2 * NPAD, DH2), src, dst)
  agg2 = agg2.reshape(NC, 2, NPAD, DH2)
  h2 = _tc2(agg2, degp, h1, Wl2, Wr2, bl2.reshape(1, D), sg2.reshape(1, D),
            b2.reshape(1, D))

  (agg3,) = _sc_agg_only(h2.reshape(2 * NPAD, DH2), src, dst)
  agg3 = agg3.reshape(NC, 2, NPAD, DH2)
  h3 = _tc3(agg3, degp, h2, Wl3, Wr3, bl3.reshape(1, DOUT),
            sg3.reshape(1, DOUT), b3.reshape(1, DOUT))

  return h3[:N]


# speedup vs baseline: 1.3164x; 1.3164x over previous
import functools

import jax
import jax.numpy as jnp
from jax import lax
from jax.experimental import pallas as pl
from jax.experimental.pallas import tpu as pltpu, tpu_sc as plsc

N = 10000
E = 320000
D = 128
DOUT = 64
EPS = 1e-5

NC = 2
NS = 16
NW = NC * NS
NPAD = 10240
RT = NPAD // NS
EW = E // NW
K = 80
NCH = EW // K
DW = 16
DH2 = D // 2


def _sc_agg(with_deg):
  mesh = plsc.VectorSubcoreMesh(core_axis_name="c", subcore_axis_name="s")
  out_type = [jax.ShapeDtypeStruct((NC, 2, NS, RT, DH2), jnp.float32)]
  if with_deg:
    out_type.append(jax.ShapeDtypeStruct((NC, NS, RT, DW), jnp.float32))
  scratch = [
      pltpu.VMEM((NCH, K), jnp.int32),
      pltpu.VMEM((NCH, K), jnp.int32),
      pltpu.VMEM((NCH, K), jnp.int32),
      pltpu.VMEM((K, DH2), jnp.float32),
      pltpu.VMEM((K, DH2), jnp.float32),
      pltpu.VMEM((K, DW), jnp.float32),
      pltpu.VMEM_SHARED((NPAD, DH2), jnp.float32),
      pltpu.VMEM_SHARED((NPAD, DW), jnp.float32),
      pltpu.SemaphoreType.DMA,
      pltpu.SemaphoreType.DMA,
      pltpu.SemaphoreType.DMA,
  ]
  if not with_deg:
    scratch = [sc for i, sc in enumerate(scratch) if i not in (5, 7, 10)]

  def body(h_hbm, src_hbm, dst_hbm, agg_out, *rest):
    if with_deg:
      (deg_out, srcb, srcb2, dstb, row0, row1, ones16, aggsh, degsh,
       gsem0, gsem1, dsem) = rest
    else:
      (srcb, srcb2, dstb, row0, row1, aggsh, gsem0, gsem1) = rest
    cid = lax.axis_index("c")
    sid = lax.axis_index("s")
    wid = sid * NC + cid
    tb = sid * RT

    pltpu.sync_copy(src_hbm.at[wid], srcb)
    pltpu.sync_copy(dst_hbm.at[wid], dstb)

    rows = (row0, row1)
    gsems = (gsem0, gsem1)

    def gather(j, b):
      pltpu.async_copy(h_hbm.at[srcb2.at[j]], rows[b], gsems[b])

    def gwait(j, b):
      pltpu.make_async_copy(h_hbm.at[srcb2.at[j]], rows[b], gsems[b]).wait()

    def scat(j, b):
      pltpu.sync_copy(rows[b], aggsh.at[dstb.at[j]], add=True)

    for half in range(2):
      def trow(j, _):
        for l in range(K // 16):
          v = srcb[j, pl.ds(l * 16, 16)]
          srcb2[j, pl.ds(l * 16, 16)] = v * 2 + half
        return 0
      lax.fori_loop(0, NCH, trow, 0)

      def zrow(i, _):
        for j in range(DH2 // 16):
          row0[i, pl.ds(j * 16, 16)] = jnp.zeros((16,), jnp.float32)
        return 0
      lax.fori_loop(0, K, zrow, 0)
      for j in range(RT // K):
        pltpu.sync_copy(row0, aggsh.at[pl.ds(tb + j * K, K)])
      if with_deg and half == 0:
        def z16row(i, _):
          ones16[i, :] = jnp.zeros((DW,), jnp.float32)
          return 0
        lax.fori_loop(0, K, z16row, 0)
        for j in range(RT // K):
          pltpu.sync_copy(ones16, degsh.at[pl.ds(tb + j * K, K)])
        def frow(i, _):
          ones16[i, :] = jnp.ones((DW,), jnp.float32)
          return 0
        lax.fori_loop(0, K, frow, 0)

      plsc.subcore_barrier()

      gather(0, 0)

      def step(t, _):
        for b in range(2):
          j = 2 * t + b
          nb = (b + 1) % 2
          gather(j + 1, nb)
          gwait(j, b)
          if with_deg and half == 0:
            pltpu.async_copy(ones16, degsh.at[dstb.at[j]], dsem, add=True)
          scat(j, b)
          if with_deg and half == 0:
            pltpu.make_async_copy(ones16, degsh.at[dstb.at[j]], dsem).wait()
        return 0
      lax.fori_loop(0, (NCH - 1) // 2, step, 0)
      jlast = NCH - 1
      gwait(jlast, 0)
      if with_deg and half == 0:
        pltpu.async_copy(ones16, degsh.at[dstb.at[jlast]], dsem, add=True)
      scat(jlast, 0)
      if with_deg and half == 0:
        pltpu.make_async_copy(ones16, degsh.at[dstb.at[jlast]], dsem).wait()

      plsc.subcore_barrier()

      pltpu.sync_copy(aggsh.at[pl.ds(tb, RT)], agg_out.at[cid].at[half].at[sid])
      if with_deg and half == 0:
        pltpu.sync_copy(degsh.at[pl.ds(tb, RT)], deg_out.at[cid].at[sid])

  return pl.kernel(
      body, out_type=out_type, mesh=mesh, scratch_types=scratch,
      compiler_params=pltpu.CompilerParams(use_tc_tiling_on_sc=False))


def _tc_dense(do, relu, normalize):
  BN = 512
  grid = NPAD // BN

  def body(aggp_ref, degp_ref, h_ref, wl_ref, wr_ref, bl_ref, sg_ref,
           bb_ref, out_ref):
    deg = degp_ref[0, :, 0:1] + degp_ref[1, :, 0:1]
    r = 1.0 / jnp.maximum(deg, 1.0)
    agg = jnp.concatenate(
        [aggp_ref[0, 0] + aggp_ref[1, 0], aggp_ref[0, 1] + aggp_ref[1, 1]],
        axis=1) * r
    z = lax.dot_general(agg, wl_ref[...], (((1,), (1,)), ((), ())),
                        preferred_element_type=jnp.float32)
    z = z + lax.dot_general(h_ref[...], wr_ref[...], (((1,), (1,)), ((), ())),
                            preferred_element_type=jnp.float32)
    z = (z + bl_ref[...]) * sg_ref[...] + bb_ref[...]
    if relu:
      z = jnp.maximum(z, 0.0)
    if normalize:
      nrm = jnp.sqrt(jnp.sum(z * z, axis=1, keepdims=True))
      z = z / jnp.maximum(nrm, 1e-12)
    out_ref[...] = z

  return pl.pallas_call(
      body,
      grid=(grid,),
      in_specs=[
          pl.BlockSpec((NC, 2, BN, DH2), lambda i: (0, 0, i, 0)),
          pl.BlockSpec((NC, BN, DW), lambda i: (0, i, 0)),
          pl.BlockSpec((BN, D), lambda i: (i, 0)),
          pl.BlockSpec((do, D), lambda i: (0, 0)),
          pl.BlockSpec((do, D), lambda i: (0, 0)),
          pl.BlockSpec((1, do), lambda i: (0, 0)),
          pl.BlockSpec((1, do), lambda i: (0, 0)),
          pl.BlockSpec((1, do), lambda i: (0, 0)),
      ],
      out_specs=pl.BlockSpec((BN, do), lambda i: (i, 0)),
      out_shape=jax.ShapeDtypeStruct((NPAD, do), jnp.float32),
  )


_sc_agg_deg = _sc_agg(with_deg=True)
_sc_agg_only = _sc_agg(with_deg=False)
_tc1 = _tc_dense(D, relu=True, normalize=False)
_tc2 = _tc_dense(D, relu=True, normalize=False)
_tc3 = _tc_dense(DOUT, relu=False, normalize=True)


def kernel(x, edge_index, Wl1, bl1, Wr1, g1, b1, Wl2, bl2, Wr2, g2, b2,
           Wl3, bl3, Wr3, g3, b3):
  src = edge_index[0].reshape(NW, NCH, K)
  dst = edge_index[1].reshape(NW, NCH, K)
  xp = jnp.pad(x, ((0, NPAD - N), (0, 0)))

  t = 1.0 / jnp.sqrt(jnp.float32(1.0 + EPS))
  sg1, sg2, sg3 = g1 * t, g2 * t, g3 * t

  agg1, deg = _sc_agg_deg(xp.reshape(2 * NPAD, DH2), src, dst)
  agg1 = agg1.reshape(NC, 2, NPAD, DH2)
  degp = deg.reshape(NC, NPAD, DW)
  h1 = _tc1(agg1, degp, xp, Wl1, Wr1, bl1.reshape(1, D), sg1.reshape(1, D),
            b1.reshape(1, D))

  (agg2,) = _sc_agg_only(h1.reshape(2 * NPAD, DH2), src, dst)
  agg2 = agg2.reshape(NC, 2, NPAD, DH2)
  h2 = _tc2(agg2, degp, h1, Wl2, Wr2, bl2.reshape(1, D), sg2.reshape(1, D),
            b2.reshape(1, D))

  (agg3,) = _sc_agg_only(h2.reshape(2 * NPAD, DH2), src, dst)
  agg3 = agg3.reshape(NC, 2, NPAD, DH2)
  h3 = _tc3(agg3, degp, h2, Wl3, Wr3, bl3.reshape(1, DOUT),
            sg3.reshape(1, DOUT), b3.reshape(1, DOUT))

  return h3[:N]
